# baseline (speedup 1.0000x reference)
import jax
import jax.numpy as jnp
from jax.experimental import pallas as pl
from jax.experimental.pallas import tpu as pltpu

_B = 256
_K = 28224
_F = 1500
_FY = 1504

_FP = 1536
_FS = 256
_NWE = 6
_KT = 2048
_KB = 14

_IT = 2048
_IB = 14


def _enc_kernel(xt_ref, *refs):
    w_refs = refs[:_NWE]
    be_ref = refs[_NWE]
    y_ref, jac_ref, acc_ref, rn2_ref = refs[_NWE + 1:]
    k = pl.program_id(0)

    @pl.when(k == 0)
    def _init():
        acc_ref[...] = jnp.zeros_like(acc_ref)
        rn2_ref[...] = jnp.zeros_like(rn2_ref)

    row = jax.lax.broadcasted_iota(jnp.int32, (_KT, 1), 0)
    lane = jax.lax.broadcasted_iota(jnp.int32, (1, _KT), 1)
    xb = jnp.where(k * _KT + row < _K, xt_ref[...], 0.0)
    for i in range(_NWE):
        wb = jnp.where(k * _KT + lane < _K, w_refs[i][...], 0.0)
        acc_ref[i * _FS:(i + 1) * _FS, :] += jax.lax.dot_general(
            wb, xb, (((1,), (0,)), ((), ())),
            preferred_element_type=jnp.float32)
        rn2_ref[i * _FS:(i + 1) * _FS, :] += jnp.sum(
            wb * wb, axis=1, keepdims=True)

    @pl.when(k == _KB - 1)
    def _finish():
        yfull = jax.nn.sigmoid(acc_ref[:_FY, :] + be_ref[...])
        frow = jax.lax.broadcasted_iota(jnp.int32, (_FY, 1), 0)
        y_ref[...] = jnp.where(frow < _F, yfull, 0.0)
        y = yfull[:_F, :]
        s = y * (1.0 - y)
        s2r = jnp.sum(s * s, axis=1, keepdims=True)
        val = jnp.sum(s2r * rn2_ref[:_F, :], keepdims=True)
        jac_ref[...] = val.reshape(1, 1, 1)


_NWD = 4
_FS2 = 376


def _dec_kernel(y_ref, *refs):
    w_refs = refs[:_NWD]
    bd_ref = refs[_NWD]
    o_ref = refs[_NWD + 1]
    acc = bd_ref[...] * 1.0
    for j in range(_NWD):
        wb = w_refs[j][...]
        if (j + 1) * _FS2 > _F:
            frow = jax.lax.broadcasted_iota(jnp.int32, (_FS2, 1), 0)
            wb = jnp.where(j * _FS2 + frow < _F, wb, 0.0)
        acc = acc + jax.lax.dot_general(
            wb, y_ref[j * _FS2:(j + 1) * _FS2, :], (((0,), (0,)), ((), ())),
            preferred_element_type=jnp.float32)
    o_ref[...] = jax.nn.sigmoid(acc)


def kernel(x, W_enc, b_enc, W_dec, b_dec):
    xt = x.T
    w_dec_t = W_dec.T

    w_enc_specs = [
        pl.BlockSpec((_FS, _KT), lambda k, i=i: (i, k)) for i in range(_NWE)
    ]
    y_enc_t, jac_parts = pl.pallas_call(
        _enc_kernel,
        grid=(_KB,),
        in_specs=[pl.BlockSpec((_KT, _B), lambda k: (k, 0))]
        + w_enc_specs
        + [pl.BlockSpec((_FY, 1), lambda k: (0, 0))],
        out_specs=[
            pl.BlockSpec((_FY, _B), lambda k: (0, 0)),
            pl.BlockSpec((1, 1, 1), lambda k: (0, 0, 0)),
        ],
        out_shape=[
            jax.ShapeDtypeStruct((_FY, _B), jnp.float32),
            jax.ShapeDtypeStruct((1, 1, 1), jnp.float32),
        ],
        scratch_shapes=[
            pltpu.VMEM((_FP, _B), jnp.float32),
            pltpu.VMEM((_FP, 1), jnp.float32),
        ],
        compiler_params=pltpu.CompilerParams(
            dimension_semantics=("arbitrary",)),
    )(xt, *([W_enc] * _NWE),
      jnp.pad(b_enc, (0, _FY - _F)).reshape(_FY, 1))

    jac_reg = jac_parts.reshape(())

    y_out_t = pl.pallas_call(
        _dec_kernel,
        grid=(_IB,),
        in_specs=[pl.BlockSpec((_FY, _B), lambda i: (0, 0))]
        + [pl.BlockSpec((_FS2, _IT), lambda i, j=j: (j, i))
           for j in range(_NWD)]
        + [pl.BlockSpec((_IT, 1), lambda i: (i, 0))],
        out_specs=pl.BlockSpec((_IT, _B), lambda i: (i, 0)),
        out_shape=jax.ShapeDtypeStruct((_K, _B), jnp.float32),
        compiler_params=pltpu.CompilerParams(
            dimension_semantics=("arbitrary",)),
    )(y_enc_t, *([w_dec_t] * _NWD), b_dec.reshape(_K, 1))

    return y_out_t.T, jac_reg


# speedup vs baseline: 1.0247x; 1.0247x over previous
import jax
import jax.numpy as jnp
from jax.experimental import pallas as pl
from jax.experimental.pallas import tpu as pltpu

_B = 256
_K = 28224
_F = 1500

_FP = 1536
_FS = 256
_NWE = 6
_KT = 2048
_KB = 14

_IT = 2048
_IB = 14


def _enc_kernel(xt_ref, *refs):
    w_refs = refs[:_NWE]
    be_ref = refs[_NWE]
    y_ref, jac_ref, acc_ref, rn2_ref = refs[_NWE + 1:]
    k = pl.program_id(0)

    @pl.when(k == 0)
    def _init():
        acc_ref[...] = jnp.zeros_like(acc_ref)
        rn2_ref[...] = jnp.zeros_like(rn2_ref)

    row = jax.lax.broadcasted_iota(jnp.int32, (_KT, 1), 0)
    lane = jax.lax.broadcasted_iota(jnp.int32, (1, _KT), 1)
    xb = jnp.where(k * _KT + row < _K, xt_ref[...], 0.0)
    for i in range(_NWE):
        wb = jnp.where(k * _KT + lane < _K, w_refs[i][...], 0.0)
        acc_ref[i * _FS:(i + 1) * _FS, :] += jax.lax.dot_general(
            wb, xb, (((1,), (0,)), ((), ())),
            preferred_element_type=jnp.float32)
        rn2_ref[i * _FS:(i + 1) * _FS, :] += jnp.sum(
            wb * wb, axis=1, keepdims=True)

    @pl.when(k == _KB - 1)
    def _finish():
        y = jax.nn.sigmoid(acc_ref[:_F, :] + be_ref[...])
        y_ref[...] = y
        s = y * (1.0 - y)
        s2r = jnp.sum(s * s, axis=1, keepdims=True)
        val = jnp.sum(s2r * rn2_ref[:_F, :], keepdims=True)
        jac_ref[...] = val.reshape(1, 1, 1)


_NWD = 2
_IS = _IT // _NWD


def _dec_kernel(y_ref, *refs):
    w_refs = refs[:_NWD]
    bd_ref = refs[_NWD]
    o_ref = refs[_NWD + 1]
    for j in range(_NWD):
        o_ref[j * _IS:(j + 1) * _IS, :] = jax.nn.sigmoid(
            jax.lax.dot_general(
                w_refs[j][...], y_ref[...], (((0,), (0,)), ((), ())),
                preferred_element_type=jnp.float32)
            + bd_ref[j * _IS:(j + 1) * _IS, :])


def kernel(x, W_enc, b_enc, W_dec, b_dec):
    xt = x.T
    w_dec_t = W_dec.T

    w_enc_specs = [
        pl.BlockSpec((_FS, _KT), lambda k, i=i: (i, k)) for i in range(_NWE)
    ]
    y_enc_t, jac_parts = pl.pallas_call(
        _enc_kernel,
        grid=(_KB,),
        in_specs=[pl.BlockSpec((_KT, _B), lambda k: (k, 0))]
        + w_enc_specs
        + [pl.BlockSpec((_F, 1), lambda k: (0, 0))],
        out_specs=[
            pl.BlockSpec((_F, _B), lambda k: (0, 0)),
            pl.BlockSpec((1, 1, 1), lambda k: (0, 0, 0)),
        ],
        out_shape=[
            jax.ShapeDtypeStruct((_F, _B), jnp.float32),
            jax.ShapeDtypeStruct((1, 1, 1), jnp.float32),
        ],
        scratch_shapes=[
            pltpu.VMEM((_FP, _B), jnp.float32),
            pltpu.VMEM((_FP, 1), jnp.float32),
        ],
        compiler_params=pltpu.CompilerParams(
            dimension_semantics=("arbitrary",)),
    )(xt, *([W_enc] * _NWE), b_enc.reshape(_F, 1))

    jac_reg = jac_parts.reshape(())

    y_out_t = pl.pallas_call(
        _dec_kernel,
        grid=(_IB,),
        in_specs=[pl.BlockSpec((_F, _B), lambda i: (0, 0))]
        + [pl.BlockSpec((_F, _IS), lambda i, j=j: (0, _NWD * i + j))
           for j in range(_NWD)]
        + [pl.BlockSpec((_IT, 1), lambda i: (i, 0))],
        out_specs=pl.BlockSpec((_IT, _B), lambda i: (i, 0)),
        out_shape=jax.ShapeDtypeStruct((_K, _B), jnp.float32),
        compiler_params=pltpu.CompilerParams(
            dimension_semantics=("arbitrary",)),
    )(y_enc_t, *([w_dec_t] * _NWD), b_dec.reshape(_K, 1))

    return y_out_t.T, jac_reg


# speedup vs baseline: 1.0255x; 1.0007x over previous
import jax
import jax.numpy as jnp
from jax.experimental import pallas as pl
from jax.experimental.pallas import tpu as pltpu

_B = 256
_K = 28224
_F = 1500

_FP = 1536
_FS = 512
_NWE = 3
_KT = 2048
_KB = 14

_IT = 2048
_IB = 14


def _enc_kernel(xt_ref, *refs):
    w_refs = refs[:_NWE]
    be_ref = refs[_NWE]
    y_ref, jac_ref, acc_ref, rn2_ref = refs[_NWE + 1:]
    k = pl.program_id(0)

    @pl.when(k == 0)
    def _init():
        acc_ref[...] = jnp.zeros_like(acc_ref)
        rn2_ref[...] = jnp.zeros_like(rn2_ref)

    row = jax.lax.broadcasted_iota(jnp.int32, (_KT, 1), 0)
    lane = jax.lax.broadcasted_iota(jnp.int32, (1, _KT), 1)
    xb = jnp.where(k * _KT + row < _K, xt_ref[...], 0.0)
    for i in range(_NWE):
        wb = jnp.where(k * _KT + lane < _K, w_refs[i][...], 0.0)
        acc_ref[i * _FS:(i + 1) * _FS, :] += jax.lax.dot_general(
            wb, xb, (((1,), (0,)), ((), ())),
            preferred_element_type=jnp.float32)
        rn2_ref[i * _FS:(i + 1) * _FS, :] += jnp.sum(
            wb * wb, axis=1, keepdims=True)

    @pl.when(k == _KB - 1)
    def _finish():
        y = jax.nn.sigmoid(acc_ref[:_F, :] + be_ref[...])
        y_ref[...] = y
        s = y * (1.0 - y)
        s2r = jnp.sum(s * s, axis=1, keepdims=True)
        val = jnp.sum(s2r * rn2_ref[:_F, :], keepdims=True)
        jac_ref[...] = val.reshape(1, 1, 1)


_NWD = 4
_IS = _IT // _NWD


def _dec_kernel(y_ref, *refs):
    w_refs = refs[:_NWD]
    bd_ref = refs[_NWD]
    o_ref = refs[_NWD + 1]
    for j in range(_NWD):
        o_ref[j * _IS:(j + 1) * _IS, :] = jax.nn.sigmoid(
            jax.lax.dot_general(
                w_refs[j][...], y_ref[...], (((0,), (0,)), ((), ())),
                preferred_element_type=jnp.float32)
            + bd_ref[j * _IS:(j + 1) * _IS, :])


def kernel(x, W_enc, b_enc, W_dec, b_dec):
    xt = x.T
    w_dec_t = W_dec.T

    w_enc_specs = [
        pl.BlockSpec((_FS, _KT), lambda k, i=i: (i, k)) for i in range(_NWE)
    ]
    y_enc_t, jac_parts = pl.pallas_call(
        _enc_kernel,
        grid=(_KB,),
        in_specs=[pl.BlockSpec((_KT, _B), lambda k: (k, 0))]
        + w_enc_specs
        + [pl.BlockSpec((_F, 1), lambda k: (0, 0))],
        out_specs=[
            pl.BlockSpec((_F, _B), lambda k: (0, 0)),
            pl.BlockSpec((1, 1, 1), lambda k: (0, 0, 0)),
        ],
        out_shape=[
            jax.ShapeDtypeStruct((_F, _B), jnp.float32),
            jax.ShapeDtypeStruct((1, 1, 1), jnp.float32),
        ],
        scratch_shapes=[
            pltpu.VMEM((_FP, _B), jnp.float32),
            pltpu.VMEM((_FP, 1), jnp.float32),
        ],
        compiler_params=pltpu.CompilerParams(
            dimension_semantics=("arbitrary",)),
    )(xt, *([W_enc] * _NWE), b_enc.reshape(_F, 1))

    jac_reg = jac_parts.reshape(())

    y_out_t = pl.pallas_call(
        _dec_kernel,
        grid=(_IB,),
        in_specs=[pl.BlockSpec((_F, _B), lambda i: (0, 0))]
        + [pl.BlockSpec((_F, _IS), lambda i, j=j: (0, _NWD * i + j))
           for j in range(_NWD)]
        + [pl.BlockSpec((_IT, 1), lambda i: (i, 0))],
        out_specs=pl.BlockSpec((_IT, _B), lambda i: (i, 0)),
        out_shape=jax.ShapeDtypeStruct((_K, _B), jnp.float32),
        compiler_params=pltpu.CompilerParams(
            dimension_semantics=("arbitrary",)),
    )(y_enc_t, *([w_dec_t] * _NWD), b_dec.reshape(_K, 1))

    return y_out_t.T, jac_reg


# speedup vs baseline: 1.0305x; 1.0049x over previous
import jax
import jax.numpy as jnp
from jax.experimental import pallas as pl
from jax.experimental.pallas import tpu as pltpu

_B = 256
_K = 28224
_F = 1500

_FP = 1536
_FS = 256
_NWE = 6
_KT = 2048
_KB = 14

_IT = 2048
_IB = 14


def _enc_kernel(xt_ref, *refs):
    w_refs = refs[:_NWE]
    be_ref = refs[_NWE]
    y_ref, jac_ref, acc_ref, rn2_ref = refs[_NWE + 1:]
    k = pl.program_id(0)

    @pl.when(k == 0)
    def _init():
        acc_ref[...] = jnp.zeros_like(acc_ref)
        rn2_ref[...] = jnp.zeros_like(rn2_ref)

    row = jax.lax.broadcasted_iota(jnp.int32, (_KT, 1), 0)
    lane = jax.lax.broadcasted_iota(jnp.int32, (1, _KT), 1)
    xb = jnp.where(k * _KT + row < _K, xt_ref[...], 0.0)
    for i in range(_NWE):
        wb = jnp.where(k * _KT + lane < _K, w_refs[i][...], 0.0)
        acc_ref[i * _FS:(i + 1) * _FS, :] += jax.lax.dot_general(
            wb, xb, (((1,), (0,)), ((), ())),
            preferred_element_type=jnp.float32)
        rn2_ref[i * _FS:(i + 1) * _FS, :] += jnp.sum(
            wb * wb, axis=1, keepdims=True)

    @pl.when(k == _KB - 1)
    def _finish():
        y = jax.nn.sigmoid(acc_ref[:_F, :] + be_ref[...])
        y_ref[...] = y
        s = y * (1.0 - y)
        s2r = jnp.sum(s * s, axis=1, keepdims=True)
        val = jnp.sum(s2r * rn2_ref[:_F, :], keepdims=True)
        jac_ref[...] = val.reshape(1, 1, 1)


_NWD = 4
_IS = _IT // _NWD


def _dec_kernel(y_ref, *refs):
    w_refs = refs[:_NWD]
    bd_ref = refs[_NWD]
    o_ref = refs[_NWD + 1]
    for j in range(_NWD):
        o_ref[j * _IS:(j + 1) * _IS, :] = jax.nn.sigmoid(
            jax.lax.dot_general(
                w_refs[j][...], y_ref[...], (((0,), (0,)), ((), ())),
                preferred_element_type=jnp.float32)
            + bd_ref[j * _IS:(j + 1) * _IS, :])


def kernel(x, W_enc, b_enc, W_dec, b_dec):
    xt = x.T
    w_dec_t = W_dec.T

    w_enc_specs = [
        pl.BlockSpec((_FS, _KT), lambda k, i=i: (i, k)) for i in range(_NWE)
    ]
    y_enc_t, jac_parts = pl.pallas_call(
        _enc_kernel,
        grid=(_KB,),
        in_specs=[pl.BlockSpec((_KT, _B), lambda k: (k, 0))]
        + w_enc_specs
        + [pl.BlockSpec((_F, 1), lambda k: (0, 0))],
        out_specs=[
            pl.BlockSpec((_F, _B), lambda k: (0, 0)),
            pl.BlockSpec((1, 1, 1), lambda k: (0, 0, 0)),
        ],
        out_shape=[
            jax.ShapeDtypeStruct((_F, _B), jnp.float32),
            jax.ShapeDtypeStruct((1, 1, 1), jnp.float32),
        ],
        scratch_shapes=[
            pltpu.VMEM((_FP, _B), jnp.float32),
            pltpu.VMEM((_FP, 1), jnp.float32),
        ],
        compiler_params=pltpu.CompilerParams(
            dimension_semantics=("arbitrary",)),
    )(xt, *([W_enc] * _NWE), b_enc.reshape(_F, 1))

    jac_reg = jac_parts.reshape(())

    y_out_t = pl.pallas_call(
        _dec_kernel,
        grid=(_IB,),
        in_specs=[pl.BlockSpec((_F, _B), lambda i: (0, 0))]
        + [pl.BlockSpec((_F, _IS), lambda i, j=j: (0, _NWD * i + j))
           for j in range(_NWD)]
        + [pl.BlockSpec((_IT, 1), lambda i: (i, 0))],
        out_specs=pl.BlockSpec((_IT, _B), lambda i: (i, 0)),
        out_shape=jax.ShapeDtypeStruct((_K, _B), jnp.float32),
        compiler_params=pltpu.CompilerParams(
            dimension_semantics=("arbitrary",)),
    )(y_enc_t, *([w_dec_t] * _NWD), b_dec.reshape(_K, 1))

    return y_out_t.T, jac_reg
